# Initial kernel scaffold; baseline (speedup 1.0000x reference)
import jax
import jax.numpy as jnp
from jax.experimental import pallas as pl


def kernel(x, enc_params, dec_params):
    raise NotImplementedError("write your pallas kernel here")


# speedup vs baseline: 7.8941x; 7.8941x over previous
import jax
import jax.numpy as jnp
from jax.experimental import pallas as pl

_T_BLOCK = 2048
_NUM_EMB = 512


def _ln(x, g, b, eps=1e-5):
    m = jnp.mean(x, axis=-1, keepdims=True)
    v = jnp.mean((x - m) ** 2, axis=-1, keepdims=True)
    return (x - m) / jnp.sqrt(v + eps) * g + b


def _first_argmin(s):
    smin = jnp.min(s, axis=1, keepdims=True)
    iota = jax.lax.broadcasted_iota(jnp.int32, s.shape, 1)
    idx = jnp.min(jnp.where(s == smin, iota, s.shape[1]), axis=1)
    onehot = (iota == idx[:, None]).astype(jnp.float32)
    return smin[:, 0], onehot


def _tables_kernel(*refs):
    cb0 = refs[0][...]
    out_ref = refs[-1]
    h = cb0
    loss = jnp.zeros((_NUM_EMB,), jnp.float32)
    pos = 1
    for _ in range(3):
        W1, b1, g1, be1, W2, b2, g2, be2, cb_ref, cbT_ref = refs[pos:pos + 10]
        pos += 10
        h = _ln(jnp.dot(h, W1[...]) + b1[...], g1[...], be1[...])
        h = jnp.maximum(h, 0.0)
        h = _ln(jnp.dot(h, W2[...]) + b2[...], g2[...], be2[...])
        cb = cb_ref[...]
        cbT = cbT_ref[...]
        s = jnp.sum(cbT * cbT, axis=0)[None, :] - 2.0 * jnp.dot(h, cbT)
        _, onehot = _first_argmin(s)
        q = jnp.dot(onehot, cb)
        loss = loss + jnp.mean((q - h) ** 2, axis=1)
        h = q
    ma = h
    r = h
    for _ in range(4):
        W1, b1, g1, be1, W2, b2, g2, be2 = refs[pos:pos + 8]
        pos += 8
        r = _ln(jnp.dot(r, W1[...]) + b1[...], g1[...], be1[...])
        r = jnp.maximum(r, 0.0)
        r = _ln(jnp.dot(r, W2[...]) + b2[...], g2[...], be2[...])
    out_ref[:, 0:20] = r
    out_ref[:, 20:52] = ma
    out_ref[:, 52:53] = loss[:, None]


def _main_kernel(x_ref, W1_ref, b1_ref, g1_ref, be1_ref, W2_ref, b2_ref,
                 g2_ref, be2_ref, cbT_ref, tab_ref, r_ref, ma_ref, loss_ref):
    i = pl.program_id(0)
    h = _ln(jnp.dot(x_ref[...], W1_ref[...]) + b1_ref[...], g1_ref[...], be1_ref[...])
    h = jnp.maximum(h, 0.0)
    h = _ln(jnp.dot(h, W2_ref[...]) + b2_ref[...], g2_ref[...], be2_ref[...])
    cbT = cbT_ref[...]
    s = jnp.sum(cbT * cbT, axis=0)[None, :] - 2.0 * jnp.dot(h, cbT)
    smin, onehot = _first_argmin(s)
    dmin = smin + jnp.sum(h * h, axis=1)
    g = jnp.dot(onehot, tab_ref[...])
    r_ref[...] = g[:, 0:20]
    ma_ref[...] = g[:, 20:52]
    part = (jnp.sum(dmin) * (1.0 / cbT.shape[0]) + jnp.sum(g[:, 52])).reshape(1, 1)

    @pl.when(i == 0)
    def _():
        loss_ref[...] = part

    @pl.when(i != 0)
    def _():
        loss_ref[...] += part


def _row(v):
    return v.reshape(1, -1)


def kernel(x, enc_params, dec_params):
    T, din = x.shape
    p0 = enc_params[0]
    cb0 = p0["codebook"]
    num_emb, dim0 = cb0.shape

    tab_inputs = [cb0]
    for p in enc_params[1:]:
        tab_inputs += [p["W1"], _row(p["b1"]), _row(p["g1"]), _row(p["be1"]),
                       p["W2"], _row(p["b2"]), _row(p["g2"]), _row(p["be2"]),
                       p["codebook"], p["codebook"].T]
    for p in dec_params:
        tab_inputs += [p["W1"], _row(p["b1"]), _row(p["g1"]), _row(p["be1"]),
                       p["W2"], _row(p["b2"]), _row(p["g2"]), _row(p["be2"])]
    table = pl.pallas_call(
        _tables_kernel,
        out_shape=jax.ShapeDtypeStruct((num_emb, 53), jnp.float32),
    )(*tab_inputs)

    bt = _T_BLOCK
    grid = (T // bt,)
    d1 = p0["W1"].shape[1]
    full = lambda shp: pl.BlockSpec(shp, lambda i: (0,) * len(shp))
    out_r, out_ma, loss = pl.pallas_call(
        _main_kernel,
        grid=grid,
        in_specs=[
            pl.BlockSpec((bt, din), lambda i: (i, 0)),
            full((din, d1)), full((1, d1)), full((1, d1)), full((1, d1)),
            full((d1, dim0)), full((1, dim0)), full((1, dim0)), full((1, dim0)),
            full((dim0, num_emb)),
            full((num_emb, 53)),
        ],
        out_specs=[
            pl.BlockSpec((bt, 20), lambda i: (i, 0)),
            pl.BlockSpec((bt, 32), lambda i: (i, 0)),
            pl.BlockSpec((1, 1), lambda i: (0, 0)),
        ],
        out_shape=[
            jax.ShapeDtypeStruct((T, 20), jnp.float32),
            jax.ShapeDtypeStruct((T, 32), jnp.float32),
            jax.ShapeDtypeStruct((1, 1), jnp.float32),
        ],
    )(x, p0["W1"], _row(p0["b1"]), _row(p0["g1"]), _row(p0["be1"]),
      p0["W2"], _row(p0["b2"]), _row(p0["g2"]), _row(p0["be2"]), cb0.T, table)

    vq_loss = (jnp.float32(1.25) / T) * loss[0, 0]
    return out_r, out_ma, vq_loss
